# Initial kernel scaffold; baseline (speedup 1.0000x reference)
import jax
import jax.numpy as jnp
from jax.experimental import pallas as pl


def kernel(embeddings, edge_input, params, edge_index):
    raise NotImplementedError("write your pallas kernel here")


# speedup vs baseline: 7.1565x; 7.1565x over previous
import functools

import jax
import jax.numpy as jnp
import numpy as np
from jax import lax
from jax.experimental import pallas as pl
from jax.experimental.pallas import tpu as pltpu
from jax.experimental.pallas import tpu_sc as plsc

_BN_C = float(1.0 / np.sqrt(1.0 + 1e-5))

_N = 10000
_NPAD = 10240
_E = 160000
_EPAD = 163840
_ET = _EPAD // 32
_CH = 128
_NCH = _ET // _CH
_ROWS_T = _NPAD // 16
_HID = 16
_BE = 1280
_BN = 1024


def _stats_body(ei_ref, out_ref):
    i = pl.program_id(0)
    x = ei_ref[...]
    s1 = jnp.pad(jnp.sum(x, axis=0), (0, 124)).reshape(1, 128)
    s2 = jnp.pad(jnp.sum(x * x, axis=0), (0, 124)).reshape(1, 128)
    contrib = jnp.concatenate([s1, s2, jnp.zeros((6, 128), jnp.float32)], axis=0)

    @pl.when(i == 0)
    def _():
        out_ref[...] = contrib

    @pl.when(i > 0)
    def _():
        out_ref[...] = out_ref[...] + contrib


def _stats_call(ei):
    return pl.pallas_call(
        _stats_body,
        grid=(_E // 8000,),
        in_specs=[pl.BlockSpec((8000, 4), lambda i: (i, 0))],
        out_specs=pl.BlockSpec((8, 128), lambda i: (0, 0)),
        out_shape=jax.ShapeDtypeStruct((8, 128), jnp.float32),
    )(ei)


def _pedge_body(stats_ref, ei_ref, w1_ref, b1_ref, g_ref, bt_ref, w2_ref,
                b2_ref, out_ref):
    ef = float(_E)
    mean = stats_ref[0, :4] / ef
    var = (stats_ref[1, :4] - ef * mean * mean) / (ef - 1.0)
    inv = lax.rsqrt(var)
    x = (ei_ref[...] - mean[None, :]) * inv[None, :]
    w1 = w1_ref[...]

    def mlp(xh):
        a = xh[:, 0:1] * w1[0:1, :] + xh[:, 1:2] * w1[1:2, :] + b1_ref[...]
        a = jnp.maximum(a, 0.0)
        a = a * (g_ref[...] * _BN_C) + bt_ref[...]
        return jnp.dot(a, w2_ref[...], preferred_element_type=jnp.float32) + b2_ref[...]

    h1 = mlp(x[:, 0:2])
    h2 = mlp(x[:, 2:4])
    num = jnp.sum(h1 * h2, axis=1)
    den = jnp.maximum(jnp.sqrt(jnp.sum(h1 * h1, axis=1))
                      * jnp.sqrt(jnp.sum(h2 * h2, axis=1)), 1e-8)
    out_ref[...] = ((num / den + 1.0) * 0.5).reshape(_BE, 1)


def _pedge_call(stats, ei, p):
    whole = lambda i: (0, 0)
    return pl.pallas_call(
        _pedge_body,
        grid=(_E // _BE,),
        in_specs=[
            pl.BlockSpec((8, 128), whole),
            pl.BlockSpec((_BE, 4), lambda i: (i, 0)),
            pl.BlockSpec((2, 128), whole),
            pl.BlockSpec((1, 128), whole),
            pl.BlockSpec((1, 128), whole),
            pl.BlockSpec((1, 128), whole),
            pl.BlockSpec((128, 128), whole),
            pl.BlockSpec((1, 128), whole),
        ],
        out_specs=pl.BlockSpec((_BE, 1), lambda i: (i, 0)),
        out_shape=jax.ShapeDtypeStruct((_E, 1), jnp.float32),
    )(stats, ei, p['eW1'], p['eb1'].reshape(1, 128), p['eg'].reshape(1, 128),
      p['ebt'].reshape(1, 128), p['eW2'], p['eb2'].reshape(1, 128))


_MESH = plsc.VectorSubcoreMesh(core_axis_name="c", subcore_axis_name="s")


def _wid():
    c = lax.axis_index("c")
    s = lax.axis_index("s")
    return c, s, c * 16 + s


@functools.partial(
    pl.kernel,
    out_type=jax.ShapeDtypeStruct((2, _NPAD), jnp.float32),
    mesh=_MESH,
    scratch_types=[
        pltpu.VMEM((_CH,), jnp.int32),
        pltpu.VMEM((_CH,), jnp.float32),
        pltpu.VMEM_SHARED((_NPAD,), jnp.float32),
    ],
)
def _deg_sc(dst_hbm, pe_hbm, out_hbm, idx_v, val_v, acc_sh):
    c, s, wid = _wid()

    def zb(i, carry):
        val_v[pl.ds(i * 16, 16)] = jnp.zeros((16,), jnp.float32)
        return carry

    lax.fori_loop(0, _CH // 16, zb, 0)
    for j in range(_ROWS_T // _CH):
        pltpu.sync_copy(val_v, acc_sh.at[pl.ds(s * _ROWS_T + j * _CH, _CH)])
    plsc.subcore_barrier()

    base = wid * _ET

    def body(k, carry):
        off = base + k * _CH
        pltpu.sync_copy(dst_hbm.at[pl.ds(off, _CH)], idx_v)
        pltpu.sync_copy(pe_hbm.at[pl.ds(off, _CH)], val_v)
        pltpu.sync_copy(val_v, acc_sh.at[idx_v], add=True)
        return carry

    lax.fori_loop(0, _NCH, body, 0)
    plsc.subcore_barrier()
    pltpu.sync_copy(acc_sh.at[pl.ds(s * _ROWS_T, _ROWS_T)],
                    out_hbm.at[c, pl.ds(s * _ROWS_T, _ROWS_T)])


@functools.partial(
    pl.kernel,
    out_type=jax.ShapeDtypeStruct((_EPAD,), jnp.float32),
    mesh=_MESH,
    scratch_types=[
        pltpu.VMEM((_NPAD,), jnp.float32),
        pltpu.VMEM((_ET,), jnp.int32),
        pltpu.VMEM((_ET,), jnp.int32),
        pltpu.VMEM((_ET,), jnp.float32),
        pltpu.VMEM((_ET,), jnp.float32),
    ],
    compiler_params=pltpu.CompilerParams(needs_layout_passes=False),
)
def _norm_sc(dis_hbm, src_hbm, dst_hbm, pe_hbm, out_hbm,
             dis_v, sidx_v, didx_v, pe_v, nrm_v):
    c, s, wid = _wid()
    base = wid * _ET
    pltpu.sync_copy(dis_hbm, dis_v)
    pltpu.sync_copy(src_hbm.at[pl.ds(base, _ET)], sidx_v)
    pltpu.sync_copy(dst_hbm.at[pl.ds(base, _ET)], didx_v)
    pltpu.sync_copy(pe_hbm.at[pl.ds(base, _ET)], pe_v)

    def body(i, carry):
        sl = pl.ds(i * 16, 16)
        a = plsc.load_gather(dis_v, [sidx_v[sl]])
        b = plsc.load_gather(dis_v, [didx_v[sl]])
        nrm_v[sl] = a * pe_v[sl] * b
        return carry

    lax.fori_loop(0, _ET // 16, body, 0)
    pltpu.sync_copy(nrm_v, out_hbm.at[pl.ds(base, _ET)])


@functools.partial(
    pl.kernel,
    out_type=jax.ShapeDtypeStruct((2, _NPAD, _HID), jnp.float32),
    mesh=_MESH,
    scratch_types=[
        pltpu.VMEM((_CH,), jnp.int32),
        pltpu.VMEM((_CH,), jnp.int32),
        pltpu.VMEM((_CH,), jnp.float32),
        pltpu.VMEM((_CH, _HID), jnp.float32),
        pltpu.VMEM_SHARED((_NPAD, _HID), jnp.float32),
        pltpu.SemaphoreType.DMA,
    ],
    compiler_params=pltpu.CompilerParams(use_tc_tiling_on_sc=False),
)
def _msg_sc(xw_hbm, src_hbm, dst_hbm, nrm_hbm, out_hbm,
            sidx_v, didx_v, nrm_v, rows_v, acc_sh, sem):
    c, s, wid = _wid()

    def zb(i, carry):
        rows_v[i] = jnp.zeros((_HID,), jnp.float32)
        return carry

    lax.fori_loop(0, _CH, zb, 0)
    for j in range(_ROWS_T // _CH):
        pltpu.sync_copy(rows_v, acc_sh.at[pl.ds(s * _ROWS_T + j * _CH, _CH)])
    plsc.subcore_barrier()

    base = wid * _ET

    def body(k, carry):
        off = base + k * _CH
        pltpu.sync_copy(src_hbm.at[pl.ds(off, _CH)], sidx_v)
        pltpu.sync_copy(dst_hbm.at[pl.ds(off, _CH)], didx_v)
        pltpu.sync_copy(nrm_hbm.at[pl.ds(off, _CH)], nrm_v)
        pltpu.async_copy(xw_hbm.at[sidx_v], rows_v, sem).wait()

        def sc(g, cc):
            nrm16 = nrm_v[pl.ds(g * 16, 16)]
            for j in range(16):
                e = g * 16 + j
                rows_v[e] = rows_v[e] * nrm16[j]
            return cc

        lax.fori_loop(0, _CH // 16, sc, 0)
        pltpu.sync_copy(rows_v, acc_sh.at[didx_v], add=True)
        return carry

    lax.fori_loop(0, _NCH, body, 0)
    plsc.subcore_barrier()
    pltpu.sync_copy(acc_sh.at[pl.ds(s * _ROWS_T, _ROWS_T)],
                    out_hbm.at[c, pl.ds(s * _ROWS_T, _ROWS_T)])


def _xw0_body(emb_ref, w_ref, degp_ref, xw_ref, dis_ref, invd_ref):
    xw_ref[...] = jnp.dot(emb_ref[...], w_ref[...],
                          preferred_element_type=jnp.float32)
    d = degp_ref[0] + degp_ref[1] + 1.0
    dis_ref[...] = lax.rsqrt(d)
    invd_ref[...] = 1.0 / d


def _xw0_call(emb_pad, w0, degp3):
    return pl.pallas_call(
        _xw0_body,
        grid=(_NPAD // _BN,),
        in_specs=[
            pl.BlockSpec((_BN, 496), lambda i: (i, 0)),
            pl.BlockSpec((496, _HID), lambda i: (0, 0)),
            pl.BlockSpec((2, _BN, 1), lambda i: (0, i, 0)),
        ],
        out_specs=[
            pl.BlockSpec((_BN, _HID), lambda i: (i, 0)),
            pl.BlockSpec((_BN, 1), lambda i: (i, 0)),
            pl.BlockSpec((_BN, 1), lambda i: (i, 0)),
        ],
        out_shape=[
            jax.ShapeDtypeStruct((_NPAD, _HID), jnp.float32),
            jax.ShapeDtypeStruct((_NPAD, 1), jnp.float32),
            jax.ShapeDtypeStruct((_NPAD, 1), jnp.float32),
        ],
    )(emb_pad, w0, degp3)


def _post_body(has_prev, has_next, part_ref, xw_ref, invd_ref, prev_ref,
               gb_ref, bg_ref, bb_ref, wn_ref, layer_ref, xwn_ref):
    conv = part_ref[0] + part_ref[1] + xw_ref[...] * invd_ref[...] + gb_ref[...]
    x = conv * (bg_ref[...] * _BN_C) + bb_ref[...]
    x = jnp.maximum(x, 0.0)
    if has_prev:
        x = x + 0.7 * prev_ref[...]
    layer_ref[...] = x
    if has_next:
        xwn_ref[...] = jnp.dot(x, wn_ref[...], preferred_element_type=jnp.float32)


def _post_call(part, xw, invd, prev, gb, bg, bb, wn, has_prev):
    whole = lambda i: (0, 0)
    return pl.pallas_call(
        functools.partial(_post_body, has_prev, True),
        grid=(_NPAD // _BN,),
        in_specs=[
            pl.BlockSpec((2, _BN, _HID), lambda i: (0, i, 0)),
            pl.BlockSpec((_BN, _HID), lambda i: (i, 0)),
            pl.BlockSpec((_BN, 1), lambda i: (i, 0)),
            pl.BlockSpec((_BN, _HID), lambda i: (i, 0)),
            pl.BlockSpec((1, _HID), whole),
            pl.BlockSpec((1, _HID), whole),
            pl.BlockSpec((1, _HID), whole),
            pl.BlockSpec((_HID, _HID), whole),
        ],
        out_specs=[
            pl.BlockSpec((_BN, _HID), lambda i: (i, 0)),
            pl.BlockSpec((_BN, _HID), lambda i: (i, 0)),
        ],
        out_shape=[
            jax.ShapeDtypeStruct((_NPAD, _HID), jnp.float32),
            jax.ShapeDtypeStruct((_NPAD, _HID), jnp.float32),
        ],
    )(part, xw, invd, prev, gb, bg, bb, wn)


def _final_body(part_ref, xw_ref, invd_ref, l0_ref, l1_ref, l2_ref,
                gb_ref, bg_ref, bb_ref, lw_ref, pw_ref, pb_ref, out_ref):
    conv = part_ref[0] + part_ref[1] + xw_ref[...] * invd_ref[...] + gb_ref[...]
    x = conv * (bg_ref[...] * _BN_C) + bb_ref[...]
    x = jnp.maximum(x, 0.0)
    l2 = l2_ref[...]
    x3 = x + 0.7 * l2
    lw = lw_ref[...]
    m = jnp.max(lw, axis=1, keepdims=True)
    e = jnp.exp(lw - m)
    w = e / jnp.sum(e, axis=1, keepdims=True)
    emb = (l0_ref[...] * w[0:1, 0:1] + l1_ref[...] * w[0:1, 1:2]
           + l2 * w[0:1, 2:3] + x3 * w[0:1, 3:4])
    out_ref[...] = jnp.dot(emb, pw_ref[...],
                           preferred_element_type=jnp.float32) + pb_ref[...]


def _final_call(part, xw, invd, l0, l1, l2, gb, bg, bb, lw, pw, pb):
    whole = lambda i: (0, 0)
    blk = lambda i: (i, 0)
    return pl.pallas_call(
        _final_body,
        grid=(_NPAD // _BN,),
        in_specs=[
            pl.BlockSpec((2, _BN, _HID), lambda i: (0, i, 0)),
            pl.BlockSpec((_BN, _HID), blk),
            pl.BlockSpec((_BN, 1), blk),
            pl.BlockSpec((_BN, _HID), blk),
            pl.BlockSpec((_BN, _HID), blk),
            pl.BlockSpec((_BN, _HID), blk),
            pl.BlockSpec((1, _HID), whole),
            pl.BlockSpec((1, _HID), whole),
            pl.BlockSpec((1, _HID), whole),
            pl.BlockSpec((1, 4), whole),
            pl.BlockSpec((_HID, 2), whole),
            pl.BlockSpec((1, 2), whole),
        ],
        out_specs=pl.BlockSpec((_BN, 2), blk),
        out_shape=jax.ShapeDtypeStruct((_NPAD, 2), jnp.float32),
    )(part, xw, invd, l0, l1, l2, gb, bg, bb, lw, pw, pb)


def kernel(embeddings, edge_input, params, edge_index):
    p = params
    stats = _stats_call(edge_input)
    p_edge = _pedge_call(stats, edge_input, p).reshape(_E)

    src_pad = jnp.pad(edge_index[0], (0, _EPAD - _E))
    dst_pad = jnp.pad(edge_index[1], (0, _EPAD - _E))
    pe_pad = jnp.pad(p_edge, (0, _EPAD - _E))

    degp = _deg_sc(dst_pad, pe_pad)
    emb_pad = jnp.pad(embeddings, ((0, _NPAD - _N), (0, 0)))
    xw0, dis, invd = _xw0_call(emb_pad, p['gW0'], degp.reshape(2, _NPAD, 1))
    norm_pad = _norm_sc(dis.reshape(_NPAD), src_pad, dst_pad, pe_pad)

    part0 = _msg_sc(xw0, src_pad, dst_pad, norm_pad)
    l0, xw1 = _post_call(part0, xw0, invd, xw0, p['gb0'].reshape(1, _HID),
                         p['bg0'].reshape(1, _HID), p['bb0'].reshape(1, _HID),
                         p['gW1'], has_prev=False)
    part1 = _msg_sc(xw1, src_pad, dst_pad, norm_pad)
    l1, xw2 = _post_call(part1, xw1, invd, l0, p['gb1'].reshape(1, _HID),
                         p['bg1'].reshape(1, _HID), p['bb1'].reshape(1, _HID),
                         p['gW2'], has_prev=True)
    part2 = _msg_sc(xw2, src_pad, dst_pad, norm_pad)
    l2, xw3 = _post_call(part2, xw2, invd, l1, p['gb2'].reshape(1, _HID),
                         p['bg2'].reshape(1, _HID), p['bb2'].reshape(1, _HID),
                         p['gW3'], has_prev=True)
    part3 = _msg_sc(xw3, src_pad, dst_pad, norm_pad)
    out = _final_call(part3, xw3, invd, l0, l1, l2, p['gb3'].reshape(1, _HID),
                      p['bg3'].reshape(1, _HID), p['bb3'].reshape(1, _HID),
                      p['lw'].reshape(1, 4), p['pW'], p['pb'].reshape(1, 2))
    return out[:_N]
